# baseline (speedup 1.0000x reference)
import functools

import jax
import jax.numpy as jnp
from jax import lax
from jax.experimental import pallas as pl
from jax.experimental.pallas import tpu as pltpu
from jax.experimental.pallas import tpu_sc as plsc

_PAD = 1024
_VOCAB = 1025
_LANES = 16
_D = 1024
_BLK = 128
_NBLK = _D // _BLK

_NC = 2
_NS = 16
_NW = _NC * _NS

_CHUNK = 16


def _sc_body(comb_hbm, idx_hbm, pos_hbm, out_hbm, idx_v, shared,
             rows0, rows1, pos0, pos1, outv0, outv1,
             sr0, sr1, sp0, sp1, ss0, ss1):
    n_rows = out_hbm.shape[1]
    t_len = pos_hbm.shape[1]
    n_src = rows0.shape[0] // _CHUNK
    g = _CHUNK * n_src
    rtot = comb_hbm.shape[1]
    rows_per_w = n_rows // _NW
    steps = rows_per_w // _CHUNK
    cid = lax.axis_index("c")
    sid = lax.axis_index("s")
    wid = sid * _NC + cid

    rows = (rows0, rows1)
    pos = (pos0, pos1)
    outv = (outv0, outv1)
    sem_r = (sr0, sr1)
    sem_p = (sp0, sp1)
    sem_s = (ss0, ss1)

    pltpu.sync_copy(
        idx_hbm.at[pl.ds(wid * rows_per_w * n_src, rows_per_w * n_src)],
        idx_v)

    stage_rows = rtot // _NS

    def fire(c, b, blk):
        pltpu.async_copy(shared.at[idx_v.at[pl.ds(c * g, g)]],
                         rows[b], sem_r[b])
        base = wid * rows_per_w + c * _CHUNK
        t0 = lax.rem(base, t_len)
        pltpu.async_copy(pos_hbm.at[blk, pl.ds(t0, _CHUNK)],
                         pos[b], sem_p[b])

    def wait_fired(b):
        pltpu.make_async_copy(shared.at[idx_v.at[pl.ds(0, g)]],
                              rows[b], sem_r[b]).wait()
        pltpu.make_async_copy(pos_hbm.at[0, pl.ds(0, _CHUNK)],
                              pos[b], sem_p[b]).wait()

    def wait_store(b):
        pltpu.make_async_copy(outv[b], out_hbm.at[0, pl.ds(0, _CHUNK)],
                              sem_s[b]).wait()

    def accumulate(b):
        def col(j, carry):
            off = j * _LANES
            for r in range(_CHUNK):
                acc = pos[b][r, pl.ds(off, _LANES)]
                for kk in range(n_src):
                    acc = acc + rows[b][r * n_src + kk, pl.ds(off, _LANES)]
                outv[b][r, pl.ds(off, _LANES)] = acc
            return carry

        lax.fori_loop(0, _BLK // _LANES, col, 0)

    def block(blk, carry0):
        pltpu.sync_copy(
            comb_hbm.at[blk, pl.ds(sid * stage_rows, stage_rows)],
            shared.at[pl.ds(sid * stage_rows, stage_rows)])
        plsc.subcore_barrier()

        fire(0, 0, blk)
        fire(1, 1, blk)

        def pair(i, carry):
            for b in range(2):
                c = 2 * i + b
                pl.when(jnp.logical_or(i >= 1, blk > 0))(
                    lambda b=b: wait_store(b))
                wait_fired(b)
                accumulate(b)
                base = wid * rows_per_w + c * _CHUNK
                pltpu.async_copy(outv[b],
                                 out_hbm.at[blk, pl.ds(base, _CHUNK)],
                                 sem_s[b])
                pl.when(i < steps // 2 - 1)(
                    lambda c=c, b=b: fire(c + 2, b, blk))
            return carry

        lax.fori_loop(0, steps // 2, pair, 0)
        plsc.subcore_barrier()
        return carry0

    lax.fori_loop(0, _NBLK, block, 0)
    wait_store(0)
    wait_store(1)


def kernel(codes, tables, pos_emb):
    B, K, T = codes.shape
    d = tables.shape[-1]
    n = B * T
    rtot = K * _VOCAB + 120

    flat = tables.reshape(K * _VOCAB, d)
    zero_base = K * _VOCAB
    comb = jnp.concatenate(
        [flat, jnp.zeros((rtot - K * _VOCAB, d), jnp.float32)], axis=0)
    comb = comb.reshape(rtot, _NBLK, _BLK).transpose(1, 0, 2)

    codes_t = codes.transpose(0, 2, 1).reshape(n, K)
    k_ar = jnp.arange(K, dtype=jnp.int32)[None, :]
    idx = jnp.where(codes_t == _PAD, zero_base + k_ar,
                    codes_t + k_ar * _VOCAB)
    idx = idx.reshape(n * K).astype(jnp.int32)

    pos3 = pos_emb[:T].reshape(T, _NBLK, _BLK).transpose(1, 0, 2)

    mesh = plsc.VectorSubcoreMesh(core_axis_name="c", subcore_axis_name="s")
    rows_per_w = n // _NW
    fn = functools.partial(
        pl.kernel,
        mesh=mesh,
        out_type=jax.ShapeDtypeStruct((_NBLK, n, _BLK), jnp.float32),
        scratch_types=[
            pltpu.VMEM((rows_per_w * K,), jnp.int32),
            pltpu.VMEM_SHARED((rtot, _BLK), jnp.float32),
            pltpu.VMEM((_CHUNK * K, _BLK), jnp.float32),
            pltpu.VMEM((_CHUNK * K, _BLK), jnp.float32),
            pltpu.VMEM((_CHUNK, _BLK), jnp.float32),
            pltpu.VMEM((_CHUNK, _BLK), jnp.float32),
            pltpu.VMEM((_CHUNK, _BLK), jnp.float32),
            pltpu.VMEM((_CHUNK, _BLK), jnp.float32),
            pltpu.SemaphoreType.DMA,
            pltpu.SemaphoreType.DMA,
            pltpu.SemaphoreType.DMA,
            pltpu.SemaphoreType.DMA,
            pltpu.SemaphoreType.DMA,
            pltpu.SemaphoreType.DMA,
        ],
    )(_sc_body)
    out = fn(comb, idx, pos3)
    return out.transpose(1, 0, 2).reshape(B, T, d)


# speedup vs baseline: 1.2453x; 1.2453x over previous
import functools

import jax
import jax.numpy as jnp
from jax import lax
from jax.experimental import pallas as pl
from jax.experimental.pallas import tpu as pltpu
from jax.experimental.pallas import tpu_sc as plsc

_PAD = 1024
_VOCAB = 1025
_LANES = 16
_D = 1024

_NC = 2
_NS = 16
_NW = _NC * _NS

_CHUNK = 4
_NBUF = 3


def _sc_body(comb_hbm, idx_hbm, pos_hbm, out_hbm, idx_v,
             rows0, rows1, rows2, pos0, pos1, pos2, outv0, outv1, outv2,
             sr0, sr1, sr2, sp0, sp1, sp2, ss0, ss1, ss2):
    n_rows, d = out_hbm.shape
    t_len = pos_hbm.shape[0]
    k = rows0.shape[0] // _CHUNK
    g = _CHUNK * k
    rows_per_w = n_rows // _NW
    steps = rows_per_w // _CHUNK
    wid = lax.axis_index("s") * _NC + lax.axis_index("c")

    rows = (rows0, rows1, rows2)
    pos = (pos0, pos1, pos2)
    outv = (outv0, outv1, outv2)
    sem_r = (sr0, sr1, sr2)
    sem_p = (sp0, sp1, sp2)
    sem_s = (ss0, ss1, ss2)

    pltpu.sync_copy(idx_hbm.at[pl.ds(wid * rows_per_w * k, rows_per_w * k)],
                    idx_v)

    def fire(c, b):
        pltpu.async_copy(comb_hbm.at[idx_v.at[pl.ds(c * g, g)]],
                         rows[b], sem_r[b])
        base = wid * rows_per_w + c * _CHUNK
        t0 = lax.rem(base, t_len)
        pltpu.async_copy(pos_hbm.at[pl.ds(t0, _CHUNK)], pos[b], sem_p[b])

    def wait_fired(b):
        pltpu.make_async_copy(comb_hbm.at[idx_v.at[pl.ds(0, g)]],
                              rows[b], sem_r[b]).wait()
        pltpu.make_async_copy(pos_hbm.at[pl.ds(0, _CHUNK)],
                              pos[b], sem_p[b]).wait()

    def wait_store(b):
        pltpu.make_async_copy(outv[b], out_hbm.at[pl.ds(0, _CHUNK)],
                              sem_s[b]).wait()

    def accumulate(b):
        def col(j, carry):
            off = j * _LANES
            for r in range(_CHUNK):
                acc = pos[b][r, pl.ds(off, _LANES)]
                for kk in range(k):
                    acc = acc + rows[b][r * k + kk, pl.ds(off, _LANES)]
                outv[b][r, pl.ds(off, _LANES)] = acc
            return carry

        lax.fori_loop(0, d // _LANES, col, 0)

    for b in range(_NBUF):
        fire(b, b)

    def ring(i, carry):
        for b in range(_NBUF):
            c = _NBUF * i + b
            pl.when(i >= 1)(lambda b=b: wait_store(b))
            wait_fired(b)
            accumulate(b)
            base = wid * rows_per_w + c * _CHUNK
            pltpu.async_copy(outv[b], out_hbm.at[pl.ds(base, _CHUNK)],
                             sem_s[b])
            pl.when(c + _NBUF < steps)(lambda c=c, b=b: fire(c + _NBUF, b))
        return carry

    full = steps // _NBUF
    lax.fori_loop(0, full, ring, 0)
    for c in range(full * _NBUF, steps):
        b = c % _NBUF
        wait_store(b)
        wait_fired(b)
        accumulate(b)
        base = wid * rows_per_w + c * _CHUNK
        pltpu.async_copy(outv[b], out_hbm.at[pl.ds(base, _CHUNK)], sem_s[b])
    for b in range(_NBUF):
        wait_store(b)


def kernel(codes, tables, pos_emb):
    B, K, T = codes.shape
    d = tables.shape[-1]
    n = B * T

    flat = tables.reshape(K * _VOCAB, d)
    zero_base = K * _VOCAB
    comb = jnp.concatenate([flat, jnp.zeros((8, d), jnp.float32)], axis=0)

    codes_t = codes.transpose(0, 2, 1).reshape(n, K)
    k_ar = jnp.arange(K, dtype=jnp.int32)[None, :]
    idx = jnp.where(codes_t == _PAD, zero_base + k_ar,
                    codes_t + k_ar * _VOCAB)
    idx = idx.reshape(n * K).astype(jnp.int32)

    mesh = plsc.VectorSubcoreMesh(core_axis_name="c", subcore_axis_name="s")
    rows_per_w = n // _NW
    fn = functools.partial(
        pl.kernel,
        mesh=mesh,
        out_type=jax.ShapeDtypeStruct((n, d), jnp.float32),
        scratch_types=(
            [pltpu.VMEM((rows_per_w * K,), jnp.int32)]
            + [pltpu.VMEM((_CHUNK * K, d), jnp.float32)] * _NBUF
            + [pltpu.VMEM((_CHUNK, d), jnp.float32)] * _NBUF
            + [pltpu.VMEM((_CHUNK, d), jnp.float32)] * _NBUF
            + [pltpu.SemaphoreType.DMA] * (3 * _NBUF)
        ),
    )(_sc_body)
    out = fn(comb, idx, pos_emb[:T])
    return out.reshape(B, T, d)


# speedup vs baseline: 1.2501x; 1.0039x over previous
import functools

import jax
import jax.numpy as jnp
from jax import lax
from jax.experimental import pallas as pl
from jax.experimental.pallas import tpu as pltpu
from jax.experimental.pallas import tpu_sc as plsc

_PAD = 1024
_VOCAB = 1025
_LANES = 16
_D = 1024

_NC = 2
_NS = 16
_NW = _NC * _NS

_CHUNK = 4
_NBUF = 2


def _sc_body(comb_hbm, idx_hbm, pos_hbm, out_hbm, idx_v,
             rows0, rows1, pos0, pos1, outv0, outv1,
             sr0, sr1, sp0, sp1, ss0, ss1):
    n_rows, d = out_hbm.shape
    t_len = pos_hbm.shape[0]
    k = rows0.shape[0] // _CHUNK
    g = _CHUNK * k
    rows_per_w = n_rows // _NW
    steps = rows_per_w // _CHUNK
    wid = lax.axis_index("s") * _NC + lax.axis_index("c")

    rows = (rows0, rows1)
    pos = (pos0, pos1)
    outv = (outv0, outv1)
    sem_r = (sr0, sr1)
    sem_p = (sp0, sp1)
    sem_s = (ss0, ss1)

    pltpu.sync_copy(idx_hbm.at[pl.ds(wid * rows_per_w * k, rows_per_w * k)],
                    idx_v)

    def fire(c, b):
        pltpu.async_copy(comb_hbm.at[idx_v.at[pl.ds(c * g, g)]],
                         rows[b], sem_r[b])
        base = wid * rows_per_w + c * _CHUNK
        t0 = lax.rem(base, t_len)
        pltpu.async_copy(pos_hbm.at[pl.ds(t0, _CHUNK)], pos[b], sem_p[b])

    def wait_fired(b):
        pltpu.make_async_copy(comb_hbm.at[idx_v.at[pl.ds(0, g)]],
                              rows[b], sem_r[b]).wait()
        pltpu.make_async_copy(pos_hbm.at[pl.ds(0, _CHUNK)],
                              pos[b], sem_p[b]).wait()

    def wait_store(b):
        pltpu.make_async_copy(outv[b], out_hbm.at[pl.ds(0, _CHUNK)],
                              sem_s[b]).wait()

    def accumulate(b):
        def col(j, carry):
            off = j * _LANES
            for r in range(_CHUNK):
                acc = pos[b][r, pl.ds(off, _LANES)]
                for kk in range(k):
                    acc = acc + rows[b][r * k + kk, pl.ds(off, _LANES)]
                outv[b][r, pl.ds(off, _LANES)] = acc
            return carry

        lax.fori_loop(0, d // _LANES, col, 0)

    for b in range(_NBUF):
        fire(b, b)

    def ring(i, carry):
        for b in range(_NBUF):
            c = _NBUF * i + b
            pl.when(i >= 1)(lambda b=b: wait_store(b))
            wait_fired(b)
            accumulate(b)
            base = wid * rows_per_w + c * _CHUNK
            pltpu.async_copy(outv[b], out_hbm.at[pl.ds(base, _CHUNK)],
                             sem_s[b])
            pl.when(c + _NBUF < steps)(lambda c=c, b=b: fire(c + _NBUF, b))
        return carry

    lax.fori_loop(0, steps // _NBUF, ring, 0)
    for b in range(_NBUF):
        wait_store(b)


def kernel(codes, tables, pos_emb):
    B, K, T = codes.shape
    d = tables.shape[-1]
    n = B * T

    flat = tables.reshape(K * _VOCAB, d)
    zero_base = K * _VOCAB
    comb = jnp.concatenate([flat, jnp.zeros((8, d), jnp.float32)], axis=0)

    codes_t = codes.transpose(0, 2, 1).reshape(n, K)
    k_ar = jnp.arange(K, dtype=jnp.int32)[None, :]
    idx = jnp.where(codes_t == _PAD, zero_base + k_ar,
                    codes_t + k_ar * _VOCAB)
    idx = idx.reshape(n * K).astype(jnp.int32)

    mesh = plsc.VectorSubcoreMesh(core_axis_name="c", subcore_axis_name="s")
    rows_per_w = n // _NW
    fn = functools.partial(
        pl.kernel,
        mesh=mesh,
        out_type=jax.ShapeDtypeStruct((n, d), jnp.float32),
        scratch_types=(
            [pltpu.VMEM((rows_per_w * K,), jnp.int32)]
            + [pltpu.VMEM((_CHUNK * K, d), jnp.float32)] * _NBUF
            + [pltpu.VMEM((_CHUNK, d), jnp.float32)] * _NBUF
            + [pltpu.VMEM((_CHUNK, d), jnp.float32)] * _NBUF
            + [pltpu.SemaphoreType.DMA] * (3 * _NBUF)
        ),
    )(_sc_body)
    out = fn(comb, idx, pos_emb[:T])
    return out.reshape(B, T, d)
